# baseline (speedup 1.0000x reference)
import functools
import jax
import jax.numpy as jnp
from jax import lax
from jax.experimental import pallas as pl
from jax.experimental.pallas import tpu as pltpu
from jax.experimental.pallas import tpu_sc as plsc

N_CLASS = 1000000
EMBED_DIM = 32
BATCH = 16384
FIELDS = 26

B = BATCH * FIELDS
NC = 2
NS = 16
NW = NC * NS
B_PER_W = B // NW
GATHER = 128
CHUNK = 4 * GATHER
N_CHUNK = B_PER_W // CHUNK
G_PER_CHUNK = CHUNK // GATHER


def _gather_body(gidx_hbm, t128_hbm, out_hbm, idx_v, rows_v, sem):
    wid = lax.axis_index("s") * NC + lax.axis_index("c")
    base = wid * B_PER_W

    def chunk_body(ci, carry):
        cbase = base + ci * CHUNK
        pltpu.sync_copy(gidx_hbm.at[pl.ds(cbase, CHUNK)], idx_v)
        copies = []
        for j in range(G_PER_CHUNK):
            copies.append(
                pltpu.async_copy(
                    t128_hbm.at[idx_v.at[pl.ds(j * GATHER, GATHER)]],
                    rows_v.at[pl.ds(j * GATHER, GATHER)],
                    sem,
                )
            )
        for c in copies:
            c.wait()
        pltpu.sync_copy(rows_v, out_hbm.at[pl.ds(cbase, CHUNK)])
        return carry

    lax.fori_loop(0, N_CHUNK, chunk_body, 0, unroll=False)


@jax.jit
def kernel(X, table):
    xi = X.astype(jnp.int32).reshape(B)
    gidx = xi >> 2
    off = xi & 3
    t128 = table.reshape(N_CLASS // 4, 128)
    mesh = plsc.VectorSubcoreMesh(core_axis_name="c", subcore_axis_name="s")
    f = functools.partial(
        pl.kernel,
        mesh=mesh,
        out_type=jax.ShapeDtypeStruct((B, 128), jnp.float32),
        scratch_types=[
            pltpu.VMEM((CHUNK,), jnp.int32),
            pltpu.VMEM((CHUNK, 128), jnp.float32),
            pltpu.SemaphoreType.DMA,
        ],
    )(_gather_body)
    pad = f(gidx, t128)
    off_b = off[:, None]
    out = jnp.where(
        off_b < 2,
        jnp.where(off_b == 0, pad[:, 0:32], pad[:, 32:64]),
        jnp.where(off_b == 2, pad[:, 64:96], pad[:, 96:128]),
    )
    return out.reshape(BATCH, FIELDS, EMBED_DIM)
